# baseline (speedup 1.0000x reference)
import functools

import jax
import jax.numpy as jnp
from jax import lax
from jax.experimental import pallas as pl
from jax.experimental.pallas import tpu as pltpu
from jax.experimental.pallas import tpu_sc as plsc

_NC = 2
_NS = 16


def _pq_body(nf_ref, w_ref, p_ref, q_ref):
    pq = jnp.dot(nf_ref[...], w_ref[...], preferred_element_type=jnp.float32)
    h = pq.shape[1] // 2
    p_ref[...] = pq[:, :h]
    q_ref[...] = pq[:, h:]


def _project_nodes(nf, wpq):
    n = nf.shape[0]
    h2 = wpq.shape[1]
    return pl.pallas_call(
        _pq_body,
        out_shape=(
            jax.ShapeDtypeStruct((n, h2 // 2), jnp.float32),
            jax.ShapeDtypeStruct((n, h2 // 2), jnp.float32),
        ),
    )(nf, wpq)


def _sc_gather(table, idx):
    e = idx.shape[0]
    d = table.shape[1]
    nw = _NC * _NS
    per_w = e // nw
    ch = 1000
    n_ch = per_w // ch
    mesh = plsc.VectorSubcoreMesh(core_axis_name="c", subcore_axis_name="s")

    @functools.partial(
        pl.kernel,
        mesh=mesh,
        out_type=jax.ShapeDtypeStruct((e, d), jnp.float32),
        scratch_types=[
            pltpu.VMEM((per_w,), jnp.int32),
            pltpu.VMEM((ch, d), jnp.float32),
            pltpu.SemaphoreType.DMA,
        ],
        compiler_params=pltpu.CompilerParams(use_tc_tiling_on_sc=False),
    )
    def k(table_hbm, idx_hbm, out_hbm, idx_v, rows_v, sem):
        wid = lax.axis_index("s") * _NC + lax.axis_index("c")
        base = wid * per_w
        pltpu.sync_copy(idx_hbm.at[pl.ds(base, per_w)], idx_v)
        for c in range(n_ch):
            pltpu.async_copy(
                table_hbm.at[idx_v.at[pl.ds(c * ch, ch)]], rows_v, sem
            ).wait()
            pltpu.sync_copy(rows_v, out_hbm.at[pl.ds(base + c * ch, ch)])

    return k(table, idx)


def _main_body(ef_ref, st_ref, el_ref, ps_ref, q_ref, wcat_ref, aux_ref,
               w2_ref, out_ref, *, deg, t_steps, h):
    f32 = jnp.float32
    b = ef_ref.shape[0]
    npb = out_ref.shape[0]
    ein = ef_ref.shape[1] // t_steps
    dt = wcat_ref.shape[0] - ein - 2 * h

    tw = aux_ref[1:2, 0:dt]
    tb = aux_ref[1:2, dt:2 * dt]
    lane_dt = lax.broadcasted_iota(jnp.int32, (1, dt), 1)
    sin_mask = lane_dt < (dt - 1)

    bf16 = jnp.bfloat16
    wcat = wcat_ref[...]
    bcat = aux_ref[0:1, :].astype(bf16)
    el = el_ref[...]

    h_e = jnp.zeros((b, h), bf16)
    c_e = jnp.zeros((b, h), f32)
    h_a = jnp.zeros((b, h), bf16)
    c_a = jnp.zeros((b, h), f32)
    hle = jnp.zeros((b, h), bf16)
    hla = jnp.zeros((b, h), bf16)

    for t in range(t_steps):
        stt = st_ref[:, t:t + 1]
        pre = stt * tw + tb
        tv = jnp.where(sin_mask, jnp.sin(pre), pre).astype(bf16)
        cat = jnp.concatenate(
            [ef_ref[:, t * ein:(t + 1) * ein], tv, h_e, h_a], axis=1)
        g = (jnp.dot(cat, wcat, preferred_element_type=f32)
             .astype(bf16) + bcat)
        i_e = jax.nn.sigmoid(g[:, 0:h])
        f_e = jax.nn.sigmoid(g[:, h:2 * h])
        g_e = jnp.tanh(g[:, 2 * h:3 * h])
        o_e = jax.nn.sigmoid(g[:, 3 * h:4 * h])
        c_e = f_e.astype(f32) * c_e + i_e.astype(f32) * g_e.astype(f32)
        h_e = o_e * jnp.tanh(c_e.astype(bf16))
        i_a = jax.nn.sigmoid(g[:, 4 * h:5 * h])
        f_a = jax.nn.sigmoid(g[:, 5 * h:6 * h])
        g_a = jnp.tanh(g[:, 6 * h:7 * h])
        o_a = jax.nn.sigmoid(g[:, 7 * h:8 * h])
        c_a = f_a.astype(f32) * c_a + i_a.astype(f32) * g_a.astype(f32)
        h_a = o_a * jnp.tanh(c_a.astype(bf16))
        sel = el == (t + 1)
        hle = jnp.where(sel, h_e, hle)
        hla = jnp.where(sel, h_a, hla)

    wat = aux_ref[2:3, 0:h]
    av = jnp.sum(hla.astype(f32) * wat, axis=1, keepdims=True)
    av = jnp.where(av >= 0, av, 0.01 * av)

    beo = aux_ref[3:4, 0:h]
    m = jnp.maximum(
        ps_ref[...]
        + jnp.dot(hle, w2_ref[:, 0:h], preferred_element_type=f32) + beo,
        0.0)

    rowi = lax.broadcasted_iota(jnp.int32, (b, deg), 0)
    coli = lax.broadcasted_iota(jnp.int32, (b, deg), 1)
    d_mask = (rowi % deg) == coli
    ad = jnp.where(d_mask, av, 0.0)
    ni = lax.broadcasted_iota(jnp.int32, (npb, b), 0)
    ji = lax.broadcasted_iota(jnp.int32, (npb, b), 1)
    seg = ((ji // deg) == ni).astype(f32)
    a_mat = jnp.dot(seg, ad, preferred_element_type=f32)

    f_at = jnp.zeros((npb, deg), f32)
    lane_d = lax.broadcasted_iota(jnp.int32, (npb, deg), 1)
    for dd in range(deg):
        zd = a_mat[:, dd:dd + 1]
        fd = jnp.sum(jnp.maximum(a_mat - zd, 0.0), axis=1, keepdims=True)
        f_at = jnp.where(lane_d == dd, fd, f_at)
    supp = f_at < 1.0
    kk = jnp.sum(supp.astype(f32), axis=1, keepdims=True)
    tau = (jnp.sum(jnp.where(supp, a_mat, 0.0), axis=1, keepdims=True)
           - 1.0) / kk
    alpha = jnp.maximum(a_mat - tau, 0.0)

    rowi2 = lax.broadcasted_iota(jnp.int32, (b, npb), 0)
    coli2 = lax.broadcasted_iota(jnp.int32, (b, npb), 1)
    segb = ((rowi2 // deg) == coli2).astype(f32)
    alpha_up = jnp.dot(segb, alpha, preferred_element_type=f32)
    alpha_flat = jnp.sum(jnp.where(d_mask, alpha_up, 0.0), axis=1,
                         keepdims=True)

    h_neigh = jnp.dot(seg, alpha_flat * m, preferred_element_type=f32)

    bno = aux_ref[4:5, 0:h]
    out_ref[...] = jnp.maximum(
        q_ref[...]
        + jnp.dot(h_neigh, w2_ref[:, h:2 * h], preferred_element_type=f32)
        + bno,
        0.0)


def kernel(node_features, edge_src, edge_features, edge_len, seq_times,
           t_w0, t_b0, t_w, t_b,
           e_wih, e_whh, e_bih, e_bhh,
           a_wih, a_whh, a_bih, a_bhh,
           w_attn, w_eout, b_eout, w_node, b_node):
    n, node_in = node_features.shape
    e, t_steps, ein = edge_features.shape
    deg = e // n
    h = e_whh.shape[1]
    dt = t_w.shape[1] + 1
    lstm_in = ein + dt

    wpq = jnp.concatenate(
        [w_eout[:, :node_in], w_node[:, :node_in]], axis=0).T

    wcat = jnp.concatenate([
        jnp.concatenate([e_wih.T, a_wih.T], axis=1),
        jnp.concatenate([e_whh.T, jnp.zeros((h, 4 * h), jnp.float32)],
                        axis=1),
        jnp.concatenate([jnp.zeros((h, 4 * h), jnp.float32), a_whh.T],
                        axis=1),
    ], axis=0).astype(jnp.bfloat16)

    bcat = jnp.concatenate([e_bih + e_bhh, a_bih + a_bhh])
    aux = jnp.zeros((8, 8 * h), jnp.float32)
    aux = aux.at[0, :].set(bcat)
    aux = aux.at[1, 0:dt].set(jnp.concatenate([t_w[0], t_w0[0]]))
    aux = aux.at[1, dt:2 * dt].set(jnp.concatenate([t_b, t_b0]))
    aux = aux.at[2, 0:h].set(w_attn[0])
    aux = aux.at[3, 0:h].set(b_eout)
    aux = aux.at[4, 0:h].set(b_node)

    w2 = jnp.concatenate(
        [w_eout[:, node_in:].T, w_node[:, node_in:].T], axis=1)

    ef2 = edge_features.reshape(e, t_steps * ein).astype(jnp.bfloat16)
    el2 = edge_len.reshape(e, 1)

    p_tab, q_tab = _project_nodes(node_features, wpq)

    p_src = _sc_gather(p_tab, edge_src)

    npb = 80
    bb = npb * deg
    grid = (n // npb,)
    body = functools.partial(_main_body, deg=deg, t_steps=t_steps, h=h)
    out = pl.pallas_call(
        body,
        grid=grid,
        in_specs=[
            pl.BlockSpec((bb, t_steps * ein), lambda i: (i, 0)),
            pl.BlockSpec((bb, t_steps), lambda i: (i, 0)),
            pl.BlockSpec((bb, 1), lambda i: (i, 0)),
            pl.BlockSpec((bb, h), lambda i: (i, 0)),
            pl.BlockSpec((npb, h), lambda i: (i, 0)),
            pl.BlockSpec((lstm_in + 2 * h, 8 * h), lambda i: (0, 0)),
            pl.BlockSpec((8, 8 * h), lambda i: (0, 0)),
            pl.BlockSpec((h, 2 * h), lambda i: (0, 0)),
        ],
        out_specs=pl.BlockSpec((npb, h), lambda i: (i, 0)),
        out_shape=jax.ShapeDtypeStruct((n, h), jnp.float32),
    )(ef2, seq_times, el2, p_src, q_tab, wcat, aux, w2)
    return out


# speedup vs baseline: 1.7895x; 1.7895x over previous
import functools

import jax
import jax.numpy as jnp
from jax import lax
from jax.experimental import pallas as pl
from jax.experimental.pallas import tpu as pltpu
from jax.experimental.pallas import tpu_sc as plsc

_NC = 2
_NS = 16


def _pq_body(nf_ref, w_ref, p_ref, q_ref):
    pq = jnp.dot(nf_ref[...], w_ref[...], preferred_element_type=jnp.float32)
    h = pq.shape[1] // 2
    p_ref[...] = pq[:, :h]
    q_ref[...] = pq[:, h:]


def _project_nodes(nf, wpq):
    n = nf.shape[0]
    h2 = wpq.shape[1]
    return pl.pallas_call(
        _pq_body,
        out_shape=(
            jax.ShapeDtypeStruct((n, h2 // 2), jnp.float32),
            jax.ShapeDtypeStruct((n, h2 // 2), jnp.float32),
        ),
    )(nf, wpq)


def _sc_gather(table, idx):
    e = idx.shape[0]
    d = table.shape[1]
    nw = _NC * _NS
    per_w = e // nw
    ch = 1000
    n_ch = per_w // ch
    mesh = plsc.VectorSubcoreMesh(core_axis_name="c", subcore_axis_name="s")

    @functools.partial(
        pl.kernel,
        mesh=mesh,
        out_type=jax.ShapeDtypeStruct((e, d), jnp.float32),
        scratch_types=[
            pltpu.VMEM((per_w,), jnp.int32),
            pltpu.VMEM((ch, d), jnp.float32),
            pltpu.SemaphoreType.DMA,
        ],
        compiler_params=pltpu.CompilerParams(use_tc_tiling_on_sc=False),
    )
    def k(table_hbm, idx_hbm, out_hbm, idx_v, rows_v, sem):
        wid = lax.axis_index("s") * _NC + lax.axis_index("c")
        base = wid * per_w
        pltpu.sync_copy(idx_hbm.at[pl.ds(base, per_w)], idx_v)
        for c in range(n_ch):
            pltpu.async_copy(
                table_hbm.at[idx_v.at[pl.ds(c * ch, ch)]], rows_v, sem
            ).wait()
            pltpu.sync_copy(rows_v, out_hbm.at[pl.ds(base + c * ch, ch)])

    return k(table, idx)


def _main_body(ef_ref, st_ref, el_ref, ps_ref, q_ref, wx_ref, whh_ref,
               aux_ref, w2_ref, out_ref, *, deg, t_steps, h, dt):
    f32 = jnp.float32
    b = ef_ref.shape[0]
    npb = out_ref.shape[0]
    gw = 8 * h

    bf16 = jnp.bfloat16
    bcat = aux_ref[0:1, :].astype(bf16)
    el = el_ref[...]

    tdt = t_steps * dt
    twall = aux_ref[5:6, 0:tdt]
    tball = aux_ref[6:7, 0:tdt]
    lane = lax.broadcasted_iota(jnp.int32, (1, tdt), 1)
    sin_mask = (lane % dt) < (dt - 1)
    pre = st_ref[...] * twall + tball
    tv_all = jnp.where(sin_mask, jnp.sin(pre), pre).astype(bf16)

    xall = jnp.concatenate([ef_ref[...], tv_all], axis=1)
    xg = jnp.dot(xall, wx_ref[...], preferred_element_type=f32)

    whh = whh_ref[...]

    h_e = jnp.zeros((b, h), bf16)
    c_e = jnp.zeros((b, h), f32)
    h_a = jnp.zeros((b, h), bf16)
    c_a = jnp.zeros((b, h), f32)
    hle = jnp.zeros((b, h), bf16)
    hla = jnp.zeros((b, h), bf16)

    for t in range(t_steps):
        acc = xg[:, t * gw:(t + 1) * gw]
        if t > 0:
            hcat = jnp.concatenate([h_e, h_a], axis=1)
            acc = acc + jnp.dot(hcat, whh, preferred_element_type=f32)
        g = acc.astype(bf16) + bcat
        i_e = jax.nn.sigmoid(g[:, 0:h])
        f_e = jax.nn.sigmoid(g[:, h:2 * h])
        g_e = jnp.tanh(g[:, 2 * h:3 * h])
        o_e = jax.nn.sigmoid(g[:, 3 * h:4 * h])
        c_e = f_e.astype(f32) * c_e + i_e.astype(f32) * g_e.astype(f32)
        h_e = o_e * jnp.tanh(c_e.astype(bf16))
        i_a = jax.nn.sigmoid(g[:, 4 * h:5 * h])
        f_a = jax.nn.sigmoid(g[:, 5 * h:6 * h])
        g_a = jnp.tanh(g[:, 6 * h:7 * h])
        o_a = jax.nn.sigmoid(g[:, 7 * h:8 * h])
        c_a = f_a.astype(f32) * c_a + i_a.astype(f32) * g_a.astype(f32)
        h_a = o_a * jnp.tanh(c_a.astype(bf16))
        sel = el == (t + 1)
        hle = jnp.where(sel, h_e, hle)
        hla = jnp.where(sel, h_a, hla)

    wat = aux_ref[2:3, 0:h]
    av = jnp.sum(hla.astype(f32) * wat, axis=1, keepdims=True)
    av = jnp.where(av >= 0, av, 0.01 * av)

    beo = aux_ref[3:4, 0:h]
    m = jnp.maximum(
        ps_ref[...]
        + jnp.dot(hle, w2_ref[:, 0:h], preferred_element_type=f32) + beo,
        0.0)

    rowi = lax.broadcasted_iota(jnp.int32, (b, deg), 0)
    coli = lax.broadcasted_iota(jnp.int32, (b, deg), 1)
    d_mask = (rowi % deg) == coli
    ad = jnp.where(d_mask, av, 0.0)
    ni = lax.broadcasted_iota(jnp.int32, (npb, b), 0)
    ji = lax.broadcasted_iota(jnp.int32, (npb, b), 1)
    seg = ((ji // deg) == ni).astype(f32)
    a_mat = jnp.dot(seg, ad, preferred_element_type=f32)

    f_at = jnp.zeros((npb, deg), f32)
    lane_d = lax.broadcasted_iota(jnp.int32, (npb, deg), 1)
    for dd in range(deg):
        zd = a_mat[:, dd:dd + 1]
        fd = jnp.sum(jnp.maximum(a_mat - zd, 0.0), axis=1, keepdims=True)
        f_at = jnp.where(lane_d == dd, fd, f_at)
    supp = f_at < 1.0
    kk = jnp.sum(supp.astype(f32), axis=1, keepdims=True)
    tau = (jnp.sum(jnp.where(supp, a_mat, 0.0), axis=1, keepdims=True)
           - 1.0) / kk
    alpha = jnp.maximum(a_mat - tau, 0.0)

    rowi2 = lax.broadcasted_iota(jnp.int32, (b, npb), 0)
    coli2 = lax.broadcasted_iota(jnp.int32, (b, npb), 1)
    segb = ((rowi2 // deg) == coli2).astype(f32)
    alpha_up = jnp.dot(segb, alpha, preferred_element_type=f32)
    alpha_flat = jnp.sum(jnp.where(d_mask, alpha_up, 0.0), axis=1,
                         keepdims=True)

    h_neigh = jnp.dot(seg, alpha_flat * m, preferred_element_type=f32)

    bno = aux_ref[4:5, 0:h]
    out_ref[...] = jnp.maximum(
        q_ref[...]
        + jnp.dot(h_neigh, w2_ref[:, h:2 * h], preferred_element_type=f32)
        + bno,
        0.0)


def kernel(node_features, edge_src, edge_features, edge_len, seq_times,
           t_w0, t_b0, t_w, t_b,
           e_wih, e_whh, e_bih, e_bhh,
           a_wih, a_whh, a_bih, a_bhh,
           w_attn, w_eout, b_eout, w_node, b_node):
    n, node_in = node_features.shape
    e, t_steps, ein = edge_features.shape
    deg = e // n
    h = e_whh.shape[1]
    dt = t_w.shape[1] + 1
    lstm_in = ein + dt

    gw = 8 * h
    wpq = jnp.concatenate(
        [w_eout[:, :node_in], w_node[:, :node_in]], axis=0).T

    wih32 = jnp.concatenate([e_wih.T, a_wih.T], axis=1)
    wx = jnp.zeros((t_steps * (ein + dt), t_steps * gw), jnp.float32)
    for t in range(t_steps):
        wx = wx.at[t * ein:(t + 1) * ein, t * gw:(t + 1) * gw].set(
            wih32[:ein])
        wx = wx.at[t_steps * ein + t * dt:t_steps * ein + (t + 1) * dt,
                   t * gw:(t + 1) * gw].set(wih32[ein:])
    wx = wx.astype(jnp.bfloat16)

    whh = jnp.concatenate([
        jnp.concatenate([e_whh.T, jnp.zeros((h, 4 * h), jnp.float32)],
                        axis=1),
        jnp.concatenate([jnp.zeros((h, 4 * h), jnp.float32), a_whh.T],
                        axis=1),
    ], axis=0).astype(jnp.bfloat16)

    bcat = jnp.concatenate([e_bih + e_bhh, a_bih + a_bhh])
    tw1 = jnp.concatenate([t_w[0], t_w0[0]])
    tb1 = jnp.concatenate([t_b, t_b0])
    aux = jnp.zeros((8, gw), jnp.float32)
    aux = aux.at[0, :].set(bcat)
    aux = aux.at[2, 0:h].set(w_attn[0])
    aux = aux.at[3, 0:h].set(b_eout)
    aux = aux.at[4, 0:h].set(b_node)
    aux = aux.at[5, 0:t_steps * dt].set(jnp.tile(tw1, t_steps))
    aux = aux.at[6, 0:t_steps * dt].set(jnp.tile(tb1, t_steps))

    w2 = jnp.concatenate(
        [w_eout[:, node_in:].T, w_node[:, node_in:].T], axis=1)

    ef2 = edge_features.reshape(e, t_steps * ein).astype(jnp.bfloat16)
    st_rep = jnp.repeat(seq_times, dt, axis=1)
    el2 = edge_len.reshape(e, 1)

    p_tab, q_tab = _project_nodes(node_features, wpq)

    p_src = _sc_gather(p_tab, edge_src)

    npb = 80
    bb = npb * deg
    grid = (n // npb,)
    body = functools.partial(_main_body, deg=deg, t_steps=t_steps, h=h,
                             dt=dt)
    out = pl.pallas_call(
        body,
        grid=grid,
        in_specs=[
            pl.BlockSpec((bb, t_steps * ein), lambda i: (i, 0)),
            pl.BlockSpec((bb, t_steps * dt), lambda i: (i, 0)),
            pl.BlockSpec((bb, 1), lambda i: (i, 0)),
            pl.BlockSpec((bb, h), lambda i: (i, 0)),
            pl.BlockSpec((npb, h), lambda i: (i, 0)),
            pl.BlockSpec((t_steps * (ein + dt), t_steps * gw),
                         lambda i: (0, 0)),
            pl.BlockSpec((2 * h, gw), lambda i: (0, 0)),
            pl.BlockSpec((8, gw), lambda i: (0, 0)),
            pl.BlockSpec((h, 2 * h), lambda i: (0, 0)),
        ],
        out_specs=pl.BlockSpec((npb, h), lambda i: (i, 0)),
        out_shape=jax.ShapeDtypeStruct((n, h), jnp.float32),
    )(ef2, st_rep, el2, p_src, q_tab, wx, whh, aux, w2)
    return out


# speedup vs baseline: 2.0666x; 1.1548x over previous
import functools

import jax
import jax.numpy as jnp
from jax import lax
from jax.experimental import pallas as pl
from jax.experimental.pallas import tpu as pltpu
from jax.experimental.pallas import tpu_sc as plsc

_NC = 2
_NS = 16


def _pq_body(nf_ref, w_ref, p_ref, q_ref):
    pq = jnp.dot(nf_ref[...], w_ref[...], preferred_element_type=jnp.float32)
    h = pq.shape[1] // 2
    p_ref[...] = pq[:, :h]
    q_ref[...] = pq[:, h:]


def _project_nodes(nf, wpq):
    n = nf.shape[0]
    h2 = wpq.shape[1]
    return pl.pallas_call(
        _pq_body,
        out_shape=(
            jax.ShapeDtypeStruct((n, h2 // 2), jnp.float32),
            jax.ShapeDtypeStruct((n, h2 // 2), jnp.float32),
        ),
    )(nf, wpq)


def _sc_gather(table, idx):
    e = idx.shape[0]
    d = table.shape[1]
    nw = _NC * _NS
    per_w = e // nw
    ch = 1000
    n_ch = per_w // ch
    mesh = plsc.VectorSubcoreMesh(core_axis_name="c", subcore_axis_name="s")

    @functools.partial(
        pl.kernel,
        mesh=mesh,
        out_type=jax.ShapeDtypeStruct((e, d), jnp.float32),
        scratch_types=[
            pltpu.VMEM((per_w,), jnp.int32),
            pltpu.VMEM((ch, d), jnp.float32),
            pltpu.SemaphoreType.DMA,
        ],
        compiler_params=pltpu.CompilerParams(use_tc_tiling_on_sc=False),
    )
    def k(table_hbm, idx_hbm, out_hbm, idx_v, rows_v, sem):
        wid = lax.axis_index("s") * _NC + lax.axis_index("c")
        base = wid * per_w
        pltpu.sync_copy(idx_hbm.at[pl.ds(base, per_w)], idx_v)
        for c in range(n_ch):
            pltpu.async_copy(
                table_hbm.at[idx_v.at[pl.ds(c * ch, ch)]], rows_v, sem
            ).wait()
            pltpu.sync_copy(rows_v, out_hbm.at[pl.ds(base + c * ch, ch)])

    return k(table, idx)


def _main_body(ef_ref, st_ref, el_ref, ps_ref, q_ref, wx_ref, whh_ref,
               aux_ref, w2_ref, out_ref, *, deg, t_steps, h, dt):
    f32 = jnp.float32
    b = ef_ref.shape[0]
    npb = out_ref.shape[0]
    gw = 8 * h

    bf16 = jnp.bfloat16
    bcat = aux_ref[0:1, :].astype(bf16)
    el = el_ref[...]

    tdt = t_steps * dt
    twall = aux_ref[5:6, 0:tdt]
    tball = aux_ref[6:7, 0:tdt]
    lane = lax.broadcasted_iota(jnp.int32, (1, tdt), 1)
    sin_mask = (lane % dt) < (dt - 1)
    pre = st_ref[...] * twall + tball
    tv_all = jnp.where(sin_mask, jnp.sin(pre), pre).astype(bf16)

    xall = jnp.concatenate([ef_ref[...], tv_all], axis=1)
    xg = jnp.dot(xall, wx_ref[...], preferred_element_type=f32)

    whh = whh_ref[...]

    h2 = jnp.zeros((b, 2 * h), bf16)
    c2 = jnp.zeros((b, 2 * h), f32)
    hl2 = jnp.zeros((b, 2 * h), bf16)

    for t in range(t_steps):
        acc = xg[:, t * gw:(t + 1) * gw]
        if t > 0:
            acc = acc + jnp.dot(h2, whh, preferred_element_type=f32)
        g = acc.astype(bf16) + bcat
        i2 = jax.nn.sigmoid(g[:, 0:2 * h])
        f2 = jax.nn.sigmoid(g[:, 2 * h:4 * h])
        g2 = jnp.tanh(g[:, 4 * h:6 * h])
        o2 = jax.nn.sigmoid(g[:, 6 * h:8 * h])
        c2 = f2.astype(f32) * c2 + (i2 * g2).astype(f32)
        h2 = o2 * jnp.tanh(c2).astype(bf16)
        sel = el == (t + 1)
        hl2 = jnp.where(sel, h2, hl2)

    hle = hl2[:, 0:h]
    hla = hl2[:, h:2 * h]

    wat = aux_ref[2:3, 0:h]
    av = jnp.sum(hla.astype(f32) * wat, axis=1, keepdims=True)
    av = jnp.where(av >= 0, av, 0.01 * av)

    beo = aux_ref[3:4, 0:h]
    m = jnp.maximum(
        ps_ref[...]
        + jnp.dot(hle, w2_ref[:, 0:h], preferred_element_type=f32) + beo,
        0.0)

    rowi = lax.broadcasted_iota(jnp.int32, (b, deg), 0)
    coli = lax.broadcasted_iota(jnp.int32, (b, deg), 1)
    d_mask = (rowi % deg) == coli
    ad = jnp.where(d_mask, av, 0.0)
    ni = lax.broadcasted_iota(jnp.int32, (npb, b), 0)
    ji = lax.broadcasted_iota(jnp.int32, (npb, b), 1)
    seg = ((ji // deg) == ni).astype(f32)
    a_mat = jnp.dot(seg, ad, preferred_element_type=f32)

    f_at = jnp.zeros((npb, deg), f32)
    lane_d = lax.broadcasted_iota(jnp.int32, (npb, deg), 1)
    for dd in range(deg):
        zd = a_mat[:, dd:dd + 1]
        fd = jnp.sum(jnp.maximum(a_mat - zd, 0.0), axis=1, keepdims=True)
        f_at = jnp.where(lane_d == dd, fd, f_at)
    supp = f_at < 1.0
    kk = jnp.sum(supp.astype(f32), axis=1, keepdims=True)
    tau = (jnp.sum(jnp.where(supp, a_mat, 0.0), axis=1, keepdims=True)
           - 1.0) / kk
    alpha = jnp.maximum(a_mat - tau, 0.0)

    rowi2 = lax.broadcasted_iota(jnp.int32, (b, npb), 0)
    coli2 = lax.broadcasted_iota(jnp.int32, (b, npb), 1)
    segb = ((rowi2 // deg) == coli2).astype(f32)
    alpha_up = jnp.dot(segb, alpha, preferred_element_type=f32)
    alpha_flat = jnp.sum(jnp.where(d_mask, alpha_up, 0.0), axis=1,
                         keepdims=True)

    h_neigh = jnp.dot(seg, alpha_flat * m, preferred_element_type=f32)

    bno = aux_ref[4:5, 0:h]
    out_ref[...] = jnp.maximum(
        q_ref[...]
        + jnp.dot(h_neigh, w2_ref[:, h:2 * h], preferred_element_type=f32)
        + bno,
        0.0)


def kernel(node_features, edge_src, edge_features, edge_len, seq_times,
           t_w0, t_b0, t_w, t_b,
           e_wih, e_whh, e_bih, e_bhh,
           a_wih, a_whh, a_bih, a_bhh,
           w_attn, w_eout, b_eout, w_node, b_node):
    n, node_in = node_features.shape
    e, t_steps, ein = edge_features.shape
    deg = e // n
    h = e_whh.shape[1]
    dt = t_w.shape[1] + 1
    lstm_in = ein + dt

    gw = 8 * h
    wpq = jnp.concatenate(
        [w_eout[:, :node_in], w_node[:, :node_in]], axis=0).T

    def _pairgates(mat):
        return jnp.concatenate(
            [mat[..., q * h:(q + 1) * h] for pair in range(4)
             for q in (pair, pair + 4)], axis=-1)

    wih32 = _pairgates(
        jnp.concatenate([e_wih.T, a_wih.T], axis=1))
    wx = jnp.zeros((t_steps * (ein + dt), t_steps * gw), jnp.float32)
    for t in range(t_steps):
        wx = wx.at[t * ein:(t + 1) * ein, t * gw:(t + 1) * gw].set(
            wih32[:ein])
        wx = wx.at[t_steps * ein + t * dt:t_steps * ein + (t + 1) * dt,
                   t * gw:(t + 1) * gw].set(wih32[ein:])
    wx = wx.astype(jnp.bfloat16)

    whh = _pairgates(jnp.concatenate([
        jnp.concatenate([e_whh.T, jnp.zeros((h, 4 * h), jnp.float32)],
                        axis=1),
        jnp.concatenate([jnp.zeros((h, 4 * h), jnp.float32), a_whh.T],
                        axis=1),
    ], axis=0)).astype(jnp.bfloat16)

    bcat = _pairgates(
        jnp.concatenate([e_bih + e_bhh, a_bih + a_bhh]))
    tw1 = jnp.concatenate([t_w[0], t_w0[0]])
    tb1 = jnp.concatenate([t_b, t_b0])
    aux = jnp.zeros((8, gw), jnp.float32)
    aux = aux.at[0, :].set(bcat)
    aux = aux.at[2, 0:h].set(w_attn[0])
    aux = aux.at[3, 0:h].set(b_eout)
    aux = aux.at[4, 0:h].set(b_node)
    aux = aux.at[5, 0:t_steps * dt].set(jnp.tile(tw1, t_steps))
    aux = aux.at[6, 0:t_steps * dt].set(jnp.tile(tb1, t_steps))

    w2 = jnp.concatenate(
        [w_eout[:, node_in:].T, w_node[:, node_in:].T], axis=1)

    ef2 = edge_features.reshape(e, t_steps * ein).astype(jnp.bfloat16)
    st_rep = jnp.repeat(seq_times, dt, axis=1)
    el2 = edge_len.reshape(e, 1)

    p_tab, q_tab = _project_nodes(node_features, wpq)

    p_src = _sc_gather(p_tab, edge_src)

    npb = 80
    bb = npb * deg
    grid = (n // npb,)
    body = functools.partial(_main_body, deg=deg, t_steps=t_steps, h=h,
                             dt=dt)
    out = pl.pallas_call(
        body,
        grid=grid,
        in_specs=[
            pl.BlockSpec((bb, t_steps * ein), lambda i: (i, 0)),
            pl.BlockSpec((bb, t_steps * dt), lambda i: (i, 0)),
            pl.BlockSpec((bb, 1), lambda i: (i, 0)),
            pl.BlockSpec((bb, h), lambda i: (i, 0)),
            pl.BlockSpec((npb, h), lambda i: (i, 0)),
            pl.BlockSpec((t_steps * (ein + dt), t_steps * gw),
                         lambda i: (0, 0)),
            pl.BlockSpec((2 * h, gw), lambda i: (0, 0)),
            pl.BlockSpec((8, gw), lambda i: (0, 0)),
            pl.BlockSpec((h, 2 * h), lambda i: (0, 0)),
        ],
        out_specs=pl.BlockSpec((npb, h), lambda i: (i, 0)),
        out_shape=jax.ShapeDtypeStruct((n, h), jnp.float32),
    )(ef2, st_rep, el2, p_src, q_tab, wx, whh, aux, w2)
    return out
